# baseline (speedup 1.0000x reference)
import functools

import jax
import jax.numpy as jnp
from jax import lax
from jax.experimental import pallas as pl
from jax.experimental.pallas import tpu as pltpu
from jax.experimental.pallas import tpu_sc as plsc

N = 10000
NPAD = 10240
E = 320000
D = 128
H = 128
G = 64
OUT = 2

NC = 2
NS = 16
NW = NC * NS
EPW = E // NW
CHUNK = 80
NCHUNKS = EPW // CHUNK
RPT = NPAD // NS
CW = 16

_MESH = plsc.VectorSubcoreMesh(core_axis_name="c", subcore_axis_name="s")


def _unpack_chunk(packed_v, j, src_q, dst_q):
    for k in range(CHUNK // 16):
        p = packed_v[j, pl.ds(k * 16, 16)]
        src_q[pl.ds(k * 16, 16)] = jnp.bitwise_and(p, 16383)
        dst_q[pl.ds(k * 16, 16)] = lax.shift_right_logical(p, 14)


def _sc_aggregate_body(x_hbm, packed_hbm, z128_hbm,
                       sums_hbm,
                       acc_sh, packed_v, srcq0, dstq0, srcq1, dstq1,
                       rows0_v, rows1_v, sem0, sem1):
    cid = lax.axis_index("c")
    sid = lax.axis_index("s")
    wid = cid * NS + sid
    rbase = sid * RPT

    pltpu.sync_copy(z128_hbm, acc_sh.at[pl.ds(rbase, RPT)])
    pltpu.sync_copy(packed_hbm.at[wid], packed_v)
    plsc.subcore_barrier()

    _unpack_chunk(packed_v, 0, srcq0, dstq0)
    pltpu.async_copy(x_hbm.at[srcq0], rows0_v, sem0)

    def pair(jj, carry):
        j = 2 * jj
        _unpack_chunk(packed_v, j + 1, srcq1, dstq1)
        pltpu.async_copy(x_hbm.at[srcq1], rows1_v, sem1)
        pltpu.make_async_copy(x_hbm.at[srcq0], rows0_v, sem0).wait()
        pltpu.sync_copy(rows0_v, acc_sh.at[dstq0], add=True)
        _unpack_chunk(packed_v, j + 2, srcq0, dstq0)
        pltpu.async_copy(x_hbm.at[srcq0], rows0_v, sem0)
        pltpu.make_async_copy(x_hbm.at[srcq1], rows1_v, sem1).wait()
        pltpu.sync_copy(rows1_v, acc_sh.at[dstq1], add=True)
        return carry

    lax.fori_loop(0, NCHUNKS // 2, pair, 0)
    pltpu.make_async_copy(x_hbm.at[srcq0], rows0_v, sem0).wait()
    pltpu.sync_copy(rows0_v, acc_sh.at[dstq0], add=True)
    plsc.subcore_barrier()

    pltpu.sync_copy(acc_sh.at[pl.ds(rbase, RPT)],
                    sums_hbm.at[cid, pl.ds(rbase, RPT)])


_sc_aggregate = pl.kernel(
    _sc_aggregate_body,
    out_type=jax.ShapeDtypeStruct((NC, NPAD, D), jnp.float32),
    mesh=_MESH,
    scratch_types=[
        pltpu.MemorySpace.VMEM_SHARED((NPAD, D), jnp.float32),
        pltpu.VMEM((NCHUNKS, CHUNK), jnp.int32),
        pltpu.VMEM((CHUNK,), jnp.int32),
        pltpu.VMEM((CHUNK,), jnp.int32),
        pltpu.VMEM((CHUNK,), jnp.int32),
        pltpu.VMEM((CHUNK,), jnp.int32),
        pltpu.VMEM((CHUNK, D), jnp.float32),
        pltpu.VMEM((CHUNK, D), jnp.float32),
        pltpu.SemaphoreType.DMA,
        pltpu.SemaphoreType.DMA,
    ],
)


ECB = 2048
NCB = (E + ECB - 1) // ECB
EPADC = 160 * ECB
NHI = NPAD // 128


def _cnt_body(dstr_ref, dstc_ref, out_ref, acc_ref):
    i = pl.program_id(0)
    nblk = pl.num_programs(0)

    @pl.when(i == 0)
    def _():
        acc_ref[...] = jnp.zeros((NHI, 128), jnp.float32)

    hi = lax.shift_right_logical(dstr_ref[0], 7)
    lo = jnp.bitwise_and(dstc_ref[0], 127)
    ihi = lax.broadcasted_iota(jnp.int32, (NHI, ECB), 0)
    ilo = lax.broadcasted_iota(jnp.int32, (ECB, 128), 1)
    oh_hi = jnp.where(ihi == hi, 1.0, 0.0).astype(jnp.bfloat16)
    oh_lo = jnp.where(ilo == lo, 1.0, 0.0).astype(jnp.bfloat16)
    acc_ref[...] += jnp.dot(oh_hi, oh_lo,
                            preferred_element_type=jnp.float32)

    @pl.when(i == nblk - 1)
    def _():
        out_ref[...] = acc_ref[...]


def _tc_counts(dst):
    dstp = jnp.concatenate(
        [dst, jnp.full((EPADC - E,), NPAD - 1, jnp.int32)])
    dstr = dstp.reshape(160, 1, ECB)
    dstc = dstp.reshape(160, ECB, 1)
    cnt2d = pl.pallas_call(
        _cnt_body,
        grid=(160,),
        in_specs=[
            pl.BlockSpec((1, 1, ECB), lambda i: (i, 0, 0)),
            pl.BlockSpec((1, ECB, 1), lambda i: (i, 0, 0)),
        ],
        out_specs=pl.BlockSpec((NHI, 128), lambda i: (0, 0)),
        out_shape=jax.ShapeDtypeStruct((NHI, 128), jnp.float32),
        scratch_shapes=[pltpu.VMEM((NHI, 128), jnp.float32)],
    )(dstr, dstc)
    return cnt2d.reshape(NPAD, 1)


def _layer_body(sp_ref, cnt_ref, wt_ref, b_ref, out_ref):
    s = sp_ref[0] + sp_ref[1]
    c = cnt_ref[...]
    mean = s / jnp.maximum(c, 1.0)
    h = jnp.dot(mean, wt_ref[...], preferred_element_type=jnp.float32)
    h = h + b_ref[...]
    h = jnp.where(c > 0.0, h, 0.0)
    out_ref[...] = jnp.maximum(h, 0.0)


def _layer(sums, cnts, wt, b, rows):
    nblk = NPAD // rows
    return pl.pallas_call(
        _layer_body,
        grid=(nblk,),
        in_specs=[
            pl.BlockSpec((NC, rows, D), lambda i: (0, i, 0)),
            pl.BlockSpec((rows, 1), lambda i: (i, 0)),
            pl.BlockSpec((D, H), lambda i: (0, 0)),
            pl.BlockSpec((1, H), lambda i: (0, 0)),
        ],
        out_specs=pl.BlockSpec((rows, H), lambda i: (i, 0)),
        out_shape=jax.ShapeDtypeStruct((NPAD, H), jnp.float32),
    )(sums, cnts, wt, b)


_NEG = -3.4e38


def _final_body(sp_ref, cnt_ref, batch_ref, wt2_ref, b2_ref,
                wtf1_ref, bf1_ref, wtf2_ref, bf2_ref,
                out_ref, pooled_ref):
    i = pl.program_id(0)
    nblk = pl.num_programs(0)

    @pl.when(i == 0)
    def _():
        pooled_ref[...] = jnp.full((G + 8, H), _NEG, jnp.float32)

    s = sp_ref[0] + sp_ref[1]
    c = cnt_ref[...]
    mean = s / jnp.maximum(c, 1.0)
    h = jnp.dot(mean, wt2_ref[...], preferred_element_type=jnp.float32)
    h = h + b2_ref[...]
    h = jnp.where(c > 0.0, h, 0.0)
    h = jnp.maximum(h, 0.0)

    b = batch_ref[...]
    g_lo = jnp.min(b)
    g_hi = jnp.max(b)

    def seg(g, carry):
        m = (b == g)
        contrib = jnp.max(jnp.where(m, h, _NEG), axis=0, keepdims=True)
        cur = pooled_ref[pl.ds(g, 1), :]
        pooled_ref[pl.ds(g, 1), :] = jnp.maximum(cur, contrib)
        return carry

    lax.fori_loop(g_lo, g_hi + 1, seg, 0)

    @pl.when(i == nblk - 1)
    def _():
        pooled = pooled_ref[0:G, :]
        z = jnp.dot(pooled, wtf1_ref[...], preferred_element_type=jnp.float32)
        z = jnp.maximum(z + bf1_ref[...], 0.0)
        logits = jnp.dot(z, wtf2_ref[...], preferred_element_type=jnp.float32)
        logits = logits + bf2_ref[...]
        l0 = logits[:, 0:1]
        l1 = logits[:, 1:2]
        m = jnp.maximum(l0, l1)
        lse = m + jnp.log(jnp.exp(l0 - m) + jnp.exp(l1 - m))
        out_ref[...] = logits - lse


def _final(sums, cnts, batch2d, wt2, b2, wtf1, bf1, wtf2, bf2, rows):
    nblk = NPAD // rows
    full = lambda i: (0, 0)
    return pl.pallas_call(
        _final_body,
        grid=(nblk,),
        in_specs=[
            pl.BlockSpec((NC, rows, D), lambda i: (0, i, 0)),
            pl.BlockSpec((rows, 1), lambda i: (i, 0)),
            pl.BlockSpec((rows, 1), lambda i: (i, 0)),
            pl.BlockSpec((H, H), full),
            pl.BlockSpec((1, H), full),
            pl.BlockSpec((H, H), full),
            pl.BlockSpec((1, H), full),
            pl.BlockSpec((H, H), full),
            pl.BlockSpec((1, H), full),
        ],
        out_specs=pl.BlockSpec((G, H), lambda i: (0, 0)),
        out_shape=jax.ShapeDtypeStruct((G, H), jnp.float32),
        scratch_shapes=[pltpu.VMEM((G + 8, H), jnp.float32)],
    )(sums, cnts, batch2d, wt2, b2, wtf1, bf1, wtf2, bf2)


def kernel(x, edge_index, batch, W1, b1, W2, b2, Wf1, bf1, Wf2, bf2):
    packed = (jnp.left_shift(edge_index[1], 14) | edge_index[0]
              ).reshape(NW, NCHUNKS, CHUNK)
    z128 = jnp.zeros((RPT, D), jnp.float32)

    cnts = _tc_counts(edge_index[1])
    sums1 = _sc_aggregate(x, packed, z128)
    h1 = _layer(sums1, cnts, W1.T, b1.reshape(1, H), rows=1024)
    sums2 = _sc_aggregate(h1, packed, z128)
    wtf2 = jnp.zeros((H, H), jnp.float32).at[:, :OUT].set(Wf2.T)
    bf2p = jnp.zeros((1, H), jnp.float32).at[0, :OUT].set(bf2)
    batchp = jnp.concatenate([batch, jnp.full((NPAD - N,), G, jnp.int32)])
    res = _final(sums2, cnts, batchp.reshape(NPAD, 1), W2.T, b2.reshape(1, H),
                 Wf1.T, bf1.reshape(1, H), wtf2, bf2p, rows=1024)
    return res[:, :OUT]


# speedup vs baseline: 1.6005x; 1.6005x over previous
import functools

import jax
import jax.numpy as jnp
from jax import lax
from jax.experimental import pallas as pl
from jax.experimental.pallas import tpu as pltpu
from jax.experimental.pallas import tpu_sc as plsc

N = 10000
NPAD = 10240
E = 320000
D = 128
H = 128
G = 64
OUT = 2

NC = 2
NS = 16
NW = NC * NS
EPW = E // NW
CHUNK = 80
NCHUNKS = EPW // CHUNK
RPT = NPAD // NS
CW = 16

_MESH = plsc.VectorSubcoreMesh(core_axis_name="c", subcore_axis_name="s")


def _unpack_chunk(packed_v, j, src_q, dst_q):
    for k in range(CHUNK // 16):
        p = packed_v[j, pl.ds(k * 16, 16)]
        src_q[pl.ds(k * 16, 16)] = jnp.bitwise_and(p, 16383)
        dst_q[pl.ds(k * 16, 16)] = lax.shift_right_logical(p, 14)


def _sc_aggregate_body(x_hbm, packed_hbm, z128_hbm,
                       sums_hbm,
                       acc_sh, packed_v, srcq0, dstq0, srcq1, dstq1,
                       rows0_v, rows1_v, sem0, sem1):
    cid = lax.axis_index("c")
    sid = lax.axis_index("s")
    wid = cid * NS + sid
    rbase = sid * RPT

    pltpu.sync_copy(z128_hbm, acc_sh.at[pl.ds(rbase, RPT)])
    pltpu.sync_copy(packed_hbm.at[wid], packed_v)
    plsc.subcore_barrier()

    _unpack_chunk(packed_v, 0, srcq0, dstq0)
    pltpu.async_copy(x_hbm.at[srcq0], rows0_v, sem0)

    def pair(jj, carry):
        j = 2 * jj
        _unpack_chunk(packed_v, j + 1, srcq1, dstq1)
        pltpu.async_copy(x_hbm.at[srcq1], rows1_v, sem1)
        pltpu.make_async_copy(x_hbm.at[srcq0], rows0_v, sem0).wait()
        pltpu.sync_copy(rows0_v, acc_sh.at[dstq0], add=True)
        _unpack_chunk(packed_v, j + 2, srcq0, dstq0)
        pltpu.async_copy(x_hbm.at[srcq0], rows0_v, sem0)
        pltpu.make_async_copy(x_hbm.at[srcq1], rows1_v, sem1).wait()
        pltpu.sync_copy(rows1_v, acc_sh.at[dstq1], add=True)
        return carry

    lax.fori_loop(0, NCHUNKS // 2, pair, 0)
    pltpu.make_async_copy(x_hbm.at[srcq0], rows0_v, sem0).wait()
    pltpu.sync_copy(rows0_v, acc_sh.at[dstq0], add=True)
    plsc.subcore_barrier()

    pltpu.sync_copy(acc_sh.at[pl.ds(rbase, RPT)],
                    sums_hbm.at[cid, pl.ds(rbase, RPT)])


_sc_aggregate = pl.kernel(
    _sc_aggregate_body,
    out_type=jax.ShapeDtypeStruct((NC, NPAD, D), jnp.float32),
    mesh=_MESH,
    scratch_types=[
        pltpu.MemorySpace.VMEM_SHARED((NPAD, D), jnp.float32),
        pltpu.VMEM((NCHUNKS, CHUNK), jnp.int32),
        pltpu.VMEM((CHUNK,), jnp.int32),
        pltpu.VMEM((CHUNK,), jnp.int32),
        pltpu.VMEM((CHUNK,), jnp.int32),
        pltpu.VMEM((CHUNK,), jnp.int32),
        pltpu.VMEM((CHUNK, D), jnp.float32),
        pltpu.VMEM((CHUNK, D), jnp.float32),
        pltpu.SemaphoreType.DMA,
        pltpu.SemaphoreType.DMA,
    ],
)


ECB = 2048
NCB = (E + ECB - 1) // ECB
EPADC = 160 * ECB
NHI = NPAD // 128


def _cnt_body(dstr_ref, out_ref, acc_ref):
    i = pl.program_id(0)
    nblk = pl.num_programs(0)

    @pl.when(i == 0)
    def _():
        acc_ref[...] = jnp.zeros((NHI, 128), jnp.float32)

    d = dstr_ref[0]
    hi = lax.shift_right_logical(d, 7)
    lo = jnp.bitwise_and(d, 127)
    ihi = lax.broadcasted_iota(jnp.int32, (NHI, ECB), 0)
    ilo = lax.broadcasted_iota(jnp.int32, (128, ECB), 0)
    oh_hi = jnp.where(ihi == hi, 1.0, 0.0).astype(jnp.bfloat16)
    oh_loT = jnp.where(ilo == lo, 1.0, 0.0).astype(jnp.bfloat16)
    acc_ref[...] += lax.dot_general(
        oh_hi, oh_loT, (((1,), (1,)), ((), ())),
        preferred_element_type=jnp.float32)

    @pl.when(i == nblk - 1)
    def _():
        out_ref[...] = acc_ref[...]


def _tc_counts(dst):
    dstp = jnp.concatenate(
        [dst, jnp.full((EPADC - E,), NPAD - 1, jnp.int32)])
    dstr = dstp.reshape(160, 1, ECB)
    cnt2d = pl.pallas_call(
        _cnt_body,
        grid=(160,),
        in_specs=[
            pl.BlockSpec((1, 1, ECB), lambda i: (i, 0, 0)),
        ],
        out_specs=pl.BlockSpec((NHI, 128), lambda i: (0, 0)),
        out_shape=jax.ShapeDtypeStruct((NHI, 128), jnp.float32),
        scratch_shapes=[pltpu.VMEM((NHI, 128), jnp.float32)],
    )(dstr)
    return cnt2d.reshape(NPAD, 1)


def _layer_body(sp_ref, cnt_ref, wt_ref, b_ref, out_ref):
    s = sp_ref[0] + sp_ref[1]
    c = cnt_ref[...]
    mean = s / jnp.maximum(c, 1.0)
    h = jnp.dot(mean, wt_ref[...], preferred_element_type=jnp.float32)
    h = h + b_ref[...]
    h = jnp.where(c > 0.0, h, 0.0)
    out_ref[...] = jnp.maximum(h, 0.0)


def _layer(sums, cnts, wt, b, rows):
    nblk = NPAD // rows
    return pl.pallas_call(
        _layer_body,
        grid=(nblk,),
        in_specs=[
            pl.BlockSpec((NC, rows, D), lambda i: (0, i, 0)),
            pl.BlockSpec((rows, 1), lambda i: (i, 0)),
            pl.BlockSpec((D, H), lambda i: (0, 0)),
            pl.BlockSpec((1, H), lambda i: (0, 0)),
        ],
        out_specs=pl.BlockSpec((rows, H), lambda i: (i, 0)),
        out_shape=jax.ShapeDtypeStruct((NPAD, H), jnp.float32),
    )(sums, cnts, wt, b)


_NEG = -3.4e38


def _final_body(sp_ref, cnt_ref, batch_ref, wt2_ref, b2_ref,
                wtf1_ref, bf1_ref, wtf2_ref, bf2_ref,
                out_ref, pooled_ref):
    i = pl.program_id(0)
    nblk = pl.num_programs(0)

    @pl.when(i == 0)
    def _():
        pooled_ref[...] = jnp.full((G + 8, H), _NEG, jnp.float32)

    s = sp_ref[0] + sp_ref[1]
    c = cnt_ref[...]
    mean = s / jnp.maximum(c, 1.0)
    h = jnp.dot(mean, wt2_ref[...], preferred_element_type=jnp.float32)
    h = h + b2_ref[...]
    h = jnp.where(c > 0.0, h, 0.0)
    h = jnp.maximum(h, 0.0)

    b = batch_ref[...]
    g_lo = jnp.min(b)
    g_hi = jnp.max(b)

    def seg(g, carry):
        m = (b == g)
        contrib = jnp.max(jnp.where(m, h, _NEG), axis=0, keepdims=True)
        cur = pooled_ref[pl.ds(g, 1), :]
        pooled_ref[pl.ds(g, 1), :] = jnp.maximum(cur, contrib)
        return carry

    lax.fori_loop(g_lo, g_hi + 1, seg, 0)

    @pl.when(i == nblk - 1)
    def _():
        pooled = pooled_ref[0:G, :]
        z = jnp.dot(pooled, wtf1_ref[...], preferred_element_type=jnp.float32)
        z = jnp.maximum(z + bf1_ref[...], 0.0)
        logits = jnp.dot(z, wtf2_ref[...], preferred_element_type=jnp.float32)
        logits = logits + bf2_ref[...]
        l0 = logits[:, 0:1]
        l1 = logits[:, 1:2]
        m = jnp.maximum(l0, l1)
        lse = m + jnp.log(jnp.exp(l0 - m) + jnp.exp(l1 - m))
        out_ref[...] = logits - lse


def _final(sums, cnts, batch2d, wt2, b2, wtf1, bf1, wtf2, bf2, rows):
    nblk = NPAD // rows
    full = lambda i: (0, 0)
    return pl.pallas_call(
        _final_body,
        grid=(nblk,),
        in_specs=[
            pl.BlockSpec((NC, rows, D), lambda i: (0, i, 0)),
            pl.BlockSpec((rows, 1), lambda i: (i, 0)),
            pl.BlockSpec((rows, 1), lambda i: (i, 0)),
            pl.BlockSpec((H, H), full),
            pl.BlockSpec((1, H), full),
            pl.BlockSpec((H, H), full),
            pl.BlockSpec((1, H), full),
            pl.BlockSpec((H, H), full),
            pl.BlockSpec((1, H), full),
        ],
        out_specs=pl.BlockSpec((G, H), lambda i: (0, 0)),
        out_shape=jax.ShapeDtypeStruct((G, H), jnp.float32),
        scratch_shapes=[pltpu.VMEM((G + 8, H), jnp.float32)],
    )(sums, cnts, batch2d, wt2, b2, wtf1, bf1, wtf2, bf2)


def kernel(x, edge_index, batch, W1, b1, W2, b2, Wf1, bf1, Wf2, bf2):
    packed = (jnp.left_shift(edge_index[1], 14) | edge_index[0]
              ).reshape(NW, NCHUNKS, CHUNK)
    z128 = jnp.zeros((RPT, D), jnp.float32)

    cnts = _tc_counts(edge_index[1])
    sums1 = _sc_aggregate(x, packed, z128)
    h1 = _layer(sums1, cnts, W1.T, b1.reshape(1, H), rows=1024)
    sums2 = _sc_aggregate(h1, packed, z128)
    wtf2 = jnp.zeros((H, H), jnp.float32).at[:, :OUT].set(Wf2.T)
    bf2p = jnp.zeros((1, H), jnp.float32).at[0, :OUT].set(bf2)
    batchp = jnp.concatenate([batch, jnp.full((NPAD - N,), G, jnp.int32)])
    res = _final(sums2, cnts, batchp.reshape(NPAD, 1), W2.T, b2.reshape(1, H),
                 Wf1.T, bf1.reshape(1, H), wtf2, bf2p, rows=1024)
    return res[:, :OUT]
